# baseline (speedup 1.0000x reference)
import functools
import math

import jax
import jax.numpy as jnp
from jax.experimental import pallas as pl
from jax.experimental.pallas import tpu as pltpu

BM = 1024
BK = 1024
BQ = 1024
BKV = 1024


def _tower_body(x_ref, adj_ref, w1_ref, b1_ref, w2_ref, b2_ref, o_ref,
                s_ref, h_ref, adjb_ref, *, nk, f1, f2):
    l = pl.program_id(0)
    i = pl.program_id(1)
    k = pl.program_id(2)
    irows = pl.ds(i * BM, BM)
    tile = (irows, pl.ds(k * BK, BK))

    @pl.when((l == 0) & (i == 0) & (k == 0))
    def _():
        s_ref[:, :f1] = jax.lax.dot_general(
            x_ref[...].astype(jnp.bfloat16), w1_ref[...].astype(jnp.bfloat16),
            (((1,), (0,)), ((), ())),
            preferred_element_type=jnp.float32).astype(jnp.bfloat16)

    @pl.when(l == 0)
    def _():
        adjb_ref[tile] = adj_ref[...].astype(jnp.bfloat16)

    def _layer0_dot(rows):
        h_ref[rows, :] = jax.nn.sigmoid(
            jax.lax.dot_general(
                adjb_ref[rows, :], s_ref[:, :f1], (((1,), (0,)), ((), ())),
                preferred_element_type=jnp.float32)
            + b1_ref[...].astype(jnp.float32)).astype(jnp.bfloat16)

    @pl.when((l == 0) & (k == 0) & (i != 0))
    def _():
        _layer0_dot(pl.ds((i - 1) * BM, BM))

    @pl.when((l == 1) & (i == 0) & (k == 0))
    def _():
        _layer0_dot(pl.ds((pl.num_programs(1) - 1) * BM, BM))
        s_ref[...] = jax.lax.dot_general(
            h_ref[...], w2_ref[...].astype(jnp.bfloat16),
            (((1,), (0,)), ((), ())),
            preferred_element_type=jnp.float32).astype(jnp.bfloat16)

    @pl.when((l == 1) & (k == 0))
    def _():
        o_ref[...] = jax.nn.sigmoid(
            jax.lax.dot_general(
                adjb_ref[irows, :], s_ref[...], (((1,), (0,)), ((), ())),
                preferred_element_type=jnp.float32)
            + b2_ref[...].astype(jnp.float32)).astype(jnp.bfloat16)


def _tower(x, adj, w1, b1, w2, b2):
    n, nf = x.shape
    f1 = w1.shape[1]
    f2 = w2.shape[1]
    nk = n // BK
    return pl.pallas_call(
        functools.partial(_tower_body, nk=nk, f1=f1, f2=f2),
        grid=(2, n // BM, nk),
        in_specs=[
            pl.BlockSpec((n, nf), lambda l, i, k: (0, 0)),
            pl.BlockSpec((BM, BK), lambda l, i, k: ((1 - l) * i, (1 - l) * k)),
            pl.BlockSpec((nf, f1), lambda l, i, k: (0, 0)),
            pl.BlockSpec((1, f1), lambda l, i, k: (0, 0)),
            pl.BlockSpec((f1, f2), lambda l, i, k: (0, 0)),
            pl.BlockSpec((1, f2), lambda l, i, k: (0, 0)),
        ],
        out_specs=pl.BlockSpec((BM, f2), lambda l, i, k: (i, 0)),
        out_shape=jax.ShapeDtypeStruct((n, f2), jnp.bfloat16),
        scratch_shapes=[
            pltpu.VMEM((n, f2), jnp.bfloat16),
            pltpu.VMEM((n, f1), jnp.bfloat16),
            pltpu.VMEM((n, n), jnp.bfloat16),
        ],
        compiler_params=pltpu.CompilerParams(
            dimension_semantics=("arbitrary", "arbitrary", "arbitrary")),
    )(x, adj, w1, b1, w2, b2)


def _xattn_body(h1_ref, h2_ref, o1_ref, o2_ref, p_ref, l1_ref, l2_ref, *,
                ni, nj, scale):
    i = pl.program_id(0)
    j = pl.program_id(1)
    irows = pl.ds(i * BQ, BQ)
    jcols = pl.ds(j * BKV, BKV)
    h1b = h1_ref[irows, :]
    h2b = h2_ref[jcols, :]

    s = jax.lax.dot_general(
        h1b, h2b, (((1,), (1,)), ((), ())),
        preferred_element_type=jnp.float32)
    p = jnp.exp(s * scale)
    p_ref[irows, jcols] = p.astype(jnp.bfloat16)
    rsum = jnp.sum(p, axis=1, keepdims=True)
    csum = jnp.sum(p, axis=0, keepdims=True)

    @pl.when(j == 0)
    def _():
        l1_ref[...] = rsum

    @pl.when(j != 0)
    def _():
        l1_ref[...] += rsum

    @pl.when(i == 0)
    def _():
        l2_ref[:, jcols] = csum

    @pl.when(i != 0)
    def _():
        l2_ref[:, jcols] += csum

    @pl.when(j == nj - 1)
    def _():
        pv = jax.lax.dot_general(
            p_ref[irows, :], h2_ref[...], (((1,), (0,)), ((), ())),
            preferred_element_type=jnp.float32)
        o1_ref[...] = h1b.astype(jnp.float32) + pv / l1_ref[...]

    @pl.when(i == ni - 1)
    def _():
        ptq = jax.lax.dot_general(
            p_ref[:, jcols], h1_ref[...], (((0,), (0,)), ((), ())),
            preferred_element_type=jnp.float32)
        l2col = l2_ref[:, jcols].reshape(-1, 1)
        o2_ref[jcols, :] = h2b.astype(jnp.float32) + ptq / l2col


def _xattn(h1, h2, scale):
    n, d = h1.shape
    ni = n // BQ
    nj = n // BKV
    return pl.pallas_call(
        functools.partial(_xattn_body, ni=ni, nj=nj, scale=scale),
        grid=(ni, nj),
        in_specs=[
            pl.BlockSpec((n, d), lambda i, j: (0, 0)),
            pl.BlockSpec((n, d), lambda i, j: (0, 0)),
        ],
        out_specs=[
            pl.BlockSpec((BQ, d), lambda i, j: (i, 0)),
            pl.BlockSpec((n, d), lambda i, j: (0, 0)),
        ],
        out_shape=[
            jax.ShapeDtypeStruct((n, d), jnp.float32),
            jax.ShapeDtypeStruct((n, d), jnp.float32),
        ],
        scratch_shapes=[
            pltpu.VMEM((n, n), jnp.bfloat16),
            pltpu.VMEM((BQ, 1), jnp.float32),
            pltpu.VMEM((1, n), jnp.float32),
        ],
        compiler_params=pltpu.CompilerParams(
            dimension_semantics=("arbitrary", "arbitrary")),
    )(h1, h2)


def kernel(x1, adj1, x2, adj2, W1, b1, W2, b2):
    b1r = b1.reshape(1, -1)
    b2r = b2.reshape(1, -1)

    h1 = _tower(x1, adj1, W1, b1r, W2, b2r)
    h2 = _tower(x2, adj2, W1, b1r, W2, b2r)
    scale = 1.0 / math.sqrt(h1.shape[1])
    o1, o2 = _xattn(h1, h2, scale)
    return (o1, o2)


# speedup vs baseline: 1.1090x; 1.1090x over previous
import functools
import math

import jax
import jax.numpy as jnp
from jax.experimental import pallas as pl
from jax.experimental.pallas import tpu as pltpu

BM = 1024
BK = 1024
BQ = 1024
BKV = 1024


def _tower_body(x_ref, adj_ref, w1_ref, b1_ref, w2_ref, b2_ref, o_ref,
                xb_ref, h_ref, adjb_ref, *, nk, f1, f2):
    l = pl.program_id(0)
    i = pl.program_id(1)
    k = pl.program_id(2)
    irows = pl.ds(i * BM, BM)
    tile = (irows, pl.ds(k * BK, BK))

    @pl.when((l == 0) & (i == 0) & (k == 0))
    def _():
        xb_ref[...] = x_ref[...].astype(jnp.bfloat16)

    @pl.when(l == 0)
    def _():
        adjb_ref[tile] = adj_ref[...].astype(jnp.bfloat16)

    @pl.when((l == 0) & (k == nk - 1))
    def _():
        t = jax.lax.dot_general(
            adjb_ref[irows, :], xb_ref[...], (((1,), (0,)), ((), ())),
            preferred_element_type=jnp.float32)
        h_ref[irows, :] = jax.nn.sigmoid(
            jax.lax.dot_general(
                t.astype(jnp.bfloat16), w1_ref[...].astype(jnp.bfloat16),
                (((1,), (0,)), ((), ())),
                preferred_element_type=jnp.float32)
            + b1_ref[...].astype(jnp.float32)).astype(jnp.bfloat16)

    @pl.when((l == 1) & (k == 0))
    def _():
        t = jax.lax.dot_general(
            adjb_ref[irows, :], h_ref[...], (((1,), (0,)), ((), ())),
            preferred_element_type=jnp.float32)
        o_ref[...] = jax.nn.sigmoid(
            jax.lax.dot_general(
                t.astype(jnp.bfloat16), w2_ref[...].astype(jnp.bfloat16),
                (((1,), (0,)), ((), ())),
                preferred_element_type=jnp.float32)
            + b2_ref[...].astype(jnp.float32)).astype(jnp.bfloat16)


def _tower(x, adj, w1, b1, w2, b2):
    n, nf = x.shape
    f1 = w1.shape[1]
    f2 = w2.shape[1]
    nk = n // BK
    return pl.pallas_call(
        functools.partial(_tower_body, nk=nk, f1=f1, f2=f2),
        grid=(2, n // BM, nk),
        in_specs=[
            pl.BlockSpec((n, nf), lambda l, i, k: (0, 0)),
            pl.BlockSpec((BM, BK), lambda l, i, k: ((1 - l) * i, (1 - l) * k)),
            pl.BlockSpec((nf, f1), lambda l, i, k: (0, 0)),
            pl.BlockSpec((1, f1), lambda l, i, k: (0, 0)),
            pl.BlockSpec((f1, f2), lambda l, i, k: (0, 0)),
            pl.BlockSpec((1, f2), lambda l, i, k: (0, 0)),
        ],
        out_specs=pl.BlockSpec((BM, f2), lambda l, i, k: (i, 0)),
        out_shape=jax.ShapeDtypeStruct((n, f2), jnp.bfloat16),
        scratch_shapes=[
            pltpu.VMEM((n, nf), jnp.bfloat16),
            pltpu.VMEM((n, f1), jnp.bfloat16),
            pltpu.VMEM((n, n), jnp.bfloat16),
        ],
        compiler_params=pltpu.CompilerParams(
            dimension_semantics=("arbitrary", "arbitrary", "arbitrary")),
    )(x, adj, w1, b1, w2, b2)


def _xattn_body(h1_ref, h2_ref, o1_ref, o2_ref, p_ref, l1_ref, l2_ref, *,
                ni, nj, scale):
    i = pl.program_id(0)
    j = pl.program_id(1)
    irows = pl.ds(i * BQ, BQ)
    jcols = pl.ds(j * BKV, BKV)
    h1b = h1_ref[irows, :]
    h2b = h2_ref[jcols, :]

    s = jax.lax.dot_general(
        h1b, h2b, (((1,), (1,)), ((), ())),
        preferred_element_type=jnp.float32)
    p = jnp.exp(s * scale)
    p_ref[irows, jcols] = p.astype(jnp.bfloat16)
    rsum = jnp.sum(p, axis=1, keepdims=True)
    csum = jnp.sum(p, axis=0, keepdims=True)

    @pl.when(j == 0)
    def _():
        l1_ref[...] = rsum

    @pl.when(j != 0)
    def _():
        l1_ref[...] += rsum

    @pl.when(i == 0)
    def _():
        l2_ref[:, jcols] = csum

    @pl.when(i != 0)
    def _():
        l2_ref[:, jcols] += csum

    @pl.when(j == nj - 1)
    def _():
        pv = jax.lax.dot_general(
            p_ref[irows, :], h2_ref[...], (((1,), (0,)), ((), ())),
            preferred_element_type=jnp.float32)
        o1_ref[...] = h1b.astype(jnp.float32) + pv / l1_ref[...]

    @pl.when(i == ni - 1)
    def _():
        ptq = jax.lax.dot_general(
            p_ref[:, jcols], h1_ref[...], (((0,), (0,)), ((), ())),
            preferred_element_type=jnp.float32)
        l2col = l2_ref[:, jcols].reshape(-1, 1)
        o2_ref[jcols, :] = h2b.astype(jnp.float32) + ptq / l2col


def _xattn(h1, h2, scale):
    n, d = h1.shape
    ni = n // BQ
    nj = n // BKV
    return pl.pallas_call(
        functools.partial(_xattn_body, ni=ni, nj=nj, scale=scale),
        grid=(ni, nj),
        in_specs=[
            pl.BlockSpec((n, d), lambda i, j: (0, 0)),
            pl.BlockSpec((n, d), lambda i, j: (0, 0)),
        ],
        out_specs=[
            pl.BlockSpec((BQ, d), lambda i, j: (i, 0)),
            pl.BlockSpec((n, d), lambda i, j: (0, 0)),
        ],
        out_shape=[
            jax.ShapeDtypeStruct((n, d), jnp.float32),
            jax.ShapeDtypeStruct((n, d), jnp.float32),
        ],
        scratch_shapes=[
            pltpu.VMEM((n, n), jnp.bfloat16),
            pltpu.VMEM((BQ, 1), jnp.float32),
            pltpu.VMEM((1, n), jnp.float32),
        ],
        compiler_params=pltpu.CompilerParams(
            dimension_semantics=("arbitrary", "arbitrary")),
    )(h1, h2)


def kernel(x1, adj1, x2, adj2, W1, b1, W2, b2):
    b1r = b1.reshape(1, -1)
    b2r = b2.reshape(1, -1)

    h1 = _tower(x1, adj1, W1, b1r, W2, b2r)
    h2 = _tower(x2, adj2, W1, b1r, W2, b2r)
    scale = 1.0 / math.sqrt(h1.shape[1])
    o1, o2 = _xattn(h1, h2, scale)
    return (o1, o2)
